# baseline (speedup 1.0000x reference)
import functools

import jax
import jax.numpy as jnp
from jax import lax
from jax.experimental import pallas as pl
from jax.experimental.pallas import tpu as pltpu
from jax.experimental.pallas import tpu_sc as plsc

N = 10000
NP = 10240
E = 320000
H = 128
OUT = 64

NC = 2
NS = 16
NW = NC * NS
K = 128
CPW = 80
QC = 16
QPAIRS = QC // 2
NQ = CPW // QC
EP = NW * CPW * K
RPS = NP // NS
ZROWS = 128

_mesh = plsc.VectorSubcoreMesh(core_axis_name="c", subcore_axis_name="s")


EPT = E // NW


@functools.partial(
    pl.kernel,
    out_type=jax.ShapeDtypeStruct((NC, NP), jnp.float32),
    mesh=_mesh,
    compiler_params=pltpu.CompilerParams(needs_layout_passes=False),
    scratch_types=[
        pltpu.VMEM((EPT,), jnp.int32),
        pltpu.VMEM((NP,), jnp.float32),
        pltpu.VMEM((RPS,), jnp.float32),
        pltpu.VMEM((RPS,), jnp.float32),
        pltpu.VMEM_SHARED((NS, NP), jnp.float32),
    ],
)
def _sc_degree(dst_hbm, out_hbm, didx, hist, tmp, dcol, hist_sh):
    c = lax.axis_index("c")
    s = lax.axis_index("s")
    w = s * NC + c
    pltpu.sync_copy(dst_hbm.at[pl.ds(w * EPT, EPT)], didx)

    def zero(i, _):
        hist[pl.ds(i * 16, 16)] = jnp.zeros((16,), jnp.float32)
        return 0

    lax.fori_loop(0, NP // 16, zero, 0)
    ones = jnp.ones((16,), jnp.float32)

    def grp(i, _):
        idx = didx[pl.ds(i * 16, 16)]
        plsc.addupdate_scatter(hist, [idx], ones)
        return 0

    lax.fori_loop(0, EPT // 16, grp, 0)
    pltpu.sync_copy(hist, hist_sh.at[s])
    plsc.subcore_barrier()

    def zcol(i, _):
        dcol[pl.ds(i * 16, 16)] = jnp.zeros((16,), jnp.float32)
        return 0

    lax.fori_loop(0, RPS // 16, zcol, 0)
    for t in range(NS):
        pltpu.sync_copy(hist_sh.at[t, pl.ds(s * RPS, RPS)], tmp)

        def acc(g, _):
            dcol[pl.ds(g * 16, 16)] = dcol[pl.ds(g * 16, 16)] + tmp[pl.ds(g * 16, 16)]
            return 0

        lax.fori_loop(0, RPS // 16, acc, 0)
    pltpu.sync_copy(dcol, out_hbm.at[c, pl.ds(s * RPS, RPS)])


@functools.partial(
    pl.kernel,
    out_type=jax.ShapeDtypeStruct((NC, NP, H), jnp.float32),
    mesh=_mesh,
    scratch_types=[
        pltpu.VMEM((QC, K), jnp.int32),
        pltpu.VMEM((QC, K), jnp.int32),
        pltpu.VMEM((QC, K), jnp.int32),
        pltpu.VMEM((QC, K), jnp.int32),
        pltpu.VMEM((K, H), jnp.float32),
        pltpu.VMEM((K, H), jnp.float32),
        pltpu.VMEM((32, H), jnp.float32),
        pltpu.VMEM_SHARED((NP, H), jnp.float32),
        pltpu.SemaphoreType.DMA,
        pltpu.SemaphoreType.DMA,
        pltpu.SemaphoreType.DMA,
    ],
)
def _sc_agg(h_hbm, src_hbm, dst_hbm, out_hbm,
            sidx0, didx0, sidx1, didx1, rows_a, rows_b, zbuf, agg_sh,
            sem_a, sem_b, sem_i):
    c = lax.axis_index("c")
    s = lax.axis_index("s")
    w = s * NC + c
    pltpu.sync_copy(src_hbm.at[pl.ds(w * CPW, QC)], sidx0)
    pltpu.sync_copy(dst_hbm.at[pl.ds(w * CPW, QC)], didx0)

    def fill_z(i, _):
        for j in range(H // 16):
            zbuf[i, pl.ds(j * 16, 16)] = jnp.zeros((16,), jnp.float32)
        return 0

    lax.fori_loop(0, 32, fill_z, 0)

    def zero_sh(i, _):
        pltpu.sync_copy(zbuf, agg_sh.at[pl.ds(s * RPS + i * 32, 32)])
        return 0

    lax.fori_loop(0, RPS // 32, zero_sh, 0)
    plsc.subcore_barrier()

    bufs = [(sidx0, didx0), (sidx1, didx1)]
    for q in range(NQ):
        sidx, didx = bufs[q % 2]
        nsidx, ndidx = bufs[(q + 1) % 2]
        if q < NQ - 1:
            off = w * CPW + (q + 1) * QC
            pltpu.async_copy(src_hbm.at[pl.ds(off, QC)], nsidx, sem_i)
            pltpu.async_copy(dst_hbm.at[pl.ds(off, QC)], ndidx, sem_i)

        def gat(i, buf, sem, sidx=sidx):
            pltpu.async_copy(h_hbm.at[sidx.at[i, pl.ds(0, 64)]], buf.at[pl.ds(0, 64)], sem)
            pltpu.async_copy(h_hbm.at[sidx.at[i, pl.ds(64, 64)]], buf.at[pl.ds(64, 64)], sem)

        def gwait(i, buf, sem, sidx=sidx):
            pltpu.make_async_copy(h_hbm.at[sidx.at[i, pl.ds(0, 64)]], buf.at[pl.ds(0, 64)], sem).wait()
            pltpu.make_async_copy(h_hbm.at[sidx.at[i, pl.ds(64, 64)]], buf.at[pl.ds(64, 64)], sem).wait()

        def pair(g, _, sidx=sidx, didx=didx, gat=gat, gwait=gwait):
            i0 = 2 * g
            gat(i0 + 1, rows_b, sem_b)
            gwait(i0, rows_a, sem_a)
            pltpu.sync_copy(rows_a, agg_sh.at[didx.at[i0]], add=True)

            @pl.when(g < QPAIRS - 1)
            def _():
                gat(i0 + 2, rows_a, sem_a)

            gwait(i0 + 1, rows_b, sem_b)
            pltpu.sync_copy(rows_b, agg_sh.at[didx.at[i0 + 1]], add=True)
            return 0

        gat(0, rows_a, sem_a)
        lax.fori_loop(0, QPAIRS, pair, 0)
        if q < NQ - 1:
            pltpu.make_async_copy(src_hbm.at[pl.ds(0, QC)], nsidx, sem_i).wait()
            pltpu.make_async_copy(dst_hbm.at[pl.ds(0, QC)], ndidx, sem_i).wait()
    plsc.subcore_barrier()
    pltpu.sync_copy(
        agg_sh.at[pl.ds(s * RPS, RPS)], out_hbm.at[c, pl.ds(s * RPS, RPS)]
    )


def _input_body(x_ref, w_ref, b_ref, o_ref):
    o_ref[...] = jnp.maximum(
        jnp.dot(x_ref[...], w_ref[...], preferred_element_type=jnp.float32)
        + b_ref[...][None, :],
        0.0,
    )


def _layer_body(p_ref, deg_ref, h_ref, wl_ref, wr_ref, bl_ref, g_ref, bt_ref, o_ref):
    agg = p_ref[0, :N] + p_ref[1, :N]
    mean = agg / jnp.maximum(deg_ref[...], 1.0)
    h = h_ref[...]
    hn = (
        jnp.dot(mean, wl_ref[...], preferred_element_type=jnp.float32)
        + bl_ref[...][None, :]
        + jnp.dot(h, wr_ref[...], preferred_element_type=jnp.float32)
    )
    mu = jnp.mean(hn, axis=0, keepdims=True)
    var = jnp.mean((hn - mu) * (hn - mu), axis=0, keepdims=True)
    hb = (hn - mu) * lax.rsqrt(var + 1e-5) * g_ref[...][None, :] + bt_ref[...][None, :]
    o_ref[...] = jnp.maximum(hb, 0.0) + h


def _final_body(
    p_ref, deg_ref, h_ref, wl_ref, wr_ref, bl_ref,
    wn1_ref, bn1_ref, wn2_ref, bn2_ref,
    wg1_ref, bg1_ref, wg2_ref, bg2_ref,
    ne_ref, ge_ref,
):
    agg = p_ref[0, :N] + p_ref[1, :N]
    mean = agg / jnp.maximum(deg_ref[...], 1.0)
    h3 = (
        jnp.dot(mean, wl_ref[...], preferred_element_type=jnp.float32)
        + bl_ref[...][None, :]
        + jnp.dot(h_ref[...], wr_ref[...], preferred_element_type=jnp.float32)
    )
    t = jnp.maximum(
        jnp.dot(h3, wn1_ref[...], preferred_element_type=jnp.float32)
        + bn1_ref[...][None, :],
        0.0,
    )
    ne = jnp.dot(t, wn2_ref[...], preferred_element_type=jnp.float32) + bn2_ref[...][None, :]
    ne_ref[...] = ne
    gm = jnp.mean(ne, axis=0, keepdims=True)
    gx = jnp.max(ne, axis=0, keepdims=True)
    g = jnp.concatenate([gm, gx], axis=1)
    gh = jnp.maximum(
        jnp.dot(g, wg1_ref[...], preferred_element_type=jnp.float32)
        + bg1_ref[...][None, :],
        0.0,
    )
    ge_ref[...] = (
        jnp.dot(gh, wg2_ref[...], preferred_element_type=jnp.float32)
        + bg2_ref[...][None, :]
    )


def kernel(x, edge_index, edge_attr, W_in, b_in, Wl0, Wr0, bl0, Wl1, Wr1, bl1,
           Wl2, Wr2, bl2, Wl3, Wr3, bl3, gamma0, beta0, gamma1, beta1,
           gamma2, beta2, Wn1, bn1, Wn2, bn2, Wg1, bg1, Wg2, bg2):
    npad = EP - E
    pad_src = (jnp.arange(npad, dtype=jnp.int32) * 37) % N
    pad_dst = N + (jnp.arange(npad, dtype=jnp.int32) % (NP - N))
    src = jnp.concatenate([edge_index[0], pad_src]).reshape(NW * CPW, K)
    dst = jnp.concatenate([edge_index[1], pad_dst]).reshape(NW * CPW, K)

    h = pl.pallas_call(
        _input_body,
        out_shape=jax.ShapeDtypeStruct((N, H), jnp.float32),
    )(x, W_in, b_in)

    dd = _sc_degree(edge_index[1])
    deg = (dd[0] + dd[1])[:N, None]

    layer = pl.pallas_call(
        _layer_body,
        out_shape=jax.ShapeDtypeStruct((N, H), jnp.float32),
    )

    for Wl, Wr, bl, gm, bt in (
        (Wl0, Wr0, bl0, gamma0, beta0),
        (Wl1, Wr1, bl1, gamma1, beta1),
        (Wl2, Wr2, bl2, gamma2, beta2),
    ):
        p = _sc_agg(h, src, dst)
        h = layer(p, deg, h, Wl, Wr, bl, gm, bt)

    p = _sc_agg(h, src, dst)
    node_emb, graph_emb = pl.pallas_call(
        _final_body,
        out_shape=(
            jax.ShapeDtypeStruct((N, OUT), jnp.float32),
            jax.ShapeDtypeStruct((1, OUT), jnp.float32),
        ),
    )(p, deg, h, Wl3, Wr3, bl3, Wn1, bn1, Wn2, bn2, Wg1, bg1, Wg2, bg2)

    return (node_emb, graph_emb.reshape(OUT))


# speedup vs baseline: 1.0023x; 1.0023x over previous
import functools

import jax
import jax.numpy as jnp
from jax import lax
from jax.experimental import pallas as pl
from jax.experimental.pallas import tpu as pltpu
from jax.experimental.pallas import tpu_sc as plsc

N = 10000
NP = 10240
E = 320000
H = 128
OUT = 64

NC = 2
NS = 16
NW = NC * NS
K = 128
CPW = 80
QC = 16
QPAIRS = QC // 2
NQ = CPW // QC
EP = NW * CPW * K
RPS = NP // NS
ZROWS = 128

_mesh = plsc.VectorSubcoreMesh(core_axis_name="c", subcore_axis_name="s")


EPT = E // NW


@functools.partial(
    pl.kernel,
    out_type=jax.ShapeDtypeStruct((NC, NP), jnp.float32),
    mesh=_mesh,
    compiler_params=pltpu.CompilerParams(needs_layout_passes=False),
    scratch_types=[
        pltpu.VMEM((EPT,), jnp.int32),
        pltpu.VMEM((NP,), jnp.float32),
        pltpu.VMEM((RPS,), jnp.float32),
        pltpu.VMEM((RPS,), jnp.float32),
        pltpu.VMEM_SHARED((NS, NP), jnp.float32),
    ],
)
def _sc_degree(dst_hbm, out_hbm, didx, hist, tmp, dcol, hist_sh):
    c = lax.axis_index("c")
    s = lax.axis_index("s")
    w = s * NC + c
    pltpu.sync_copy(dst_hbm.at[pl.ds(w * EPT, EPT)], didx)

    def zero(i, _):
        hist[pl.ds(i * 16, 16)] = jnp.zeros((16,), jnp.float32)
        return 0

    lax.fori_loop(0, NP // 16, zero, 0)
    ones = jnp.ones((16,), jnp.float32)

    def grp(i, _):
        idx = didx[pl.ds(i * 16, 16)]
        plsc.addupdate_scatter(hist, [idx], ones)
        return 0

    lax.fori_loop(0, EPT // 16, grp, 0)
    pltpu.sync_copy(hist, hist_sh.at[s])
    plsc.subcore_barrier()

    def zcol(i, _):
        dcol[pl.ds(i * 16, 16)] = jnp.zeros((16,), jnp.float32)
        return 0

    lax.fori_loop(0, RPS // 16, zcol, 0)
    for t in range(NS):
        pltpu.sync_copy(hist_sh.at[t, pl.ds(s * RPS, RPS)], tmp)

        def acc(g, _):
            dcol[pl.ds(g * 16, 16)] = dcol[pl.ds(g * 16, 16)] + tmp[pl.ds(g * 16, 16)]
            return 0

        lax.fori_loop(0, RPS // 16, acc, 0)
    pltpu.sync_copy(dcol, out_hbm.at[c, pl.ds(s * RPS, RPS)])


@functools.partial(
    pl.kernel,
    out_type=jax.ShapeDtypeStruct((NC, NP, H), jnp.float32),
    mesh=_mesh,
    scratch_types=[
        pltpu.VMEM((QC, K), jnp.int32),
        pltpu.VMEM((QC, K), jnp.int32),
        pltpu.VMEM((QC, K), jnp.int32),
        pltpu.VMEM((QC, K), jnp.int32),
        pltpu.VMEM((K, H), jnp.float32),
        pltpu.VMEM((K, H), jnp.float32),
        pltpu.VMEM((32, H), jnp.float32),
        pltpu.VMEM_SHARED((NP, H), jnp.float32),
        pltpu.SemaphoreType.DMA,
        pltpu.SemaphoreType.DMA,
        pltpu.SemaphoreType.DMA,
    ],
)
def _sc_agg(h_hbm, src_hbm, dst_hbm, out_hbm,
            sidx0, didx0, sidx1, didx1, rows_a, rows_b, zbuf, agg_sh,
            sem_a, sem_b, sem_i):
    c = lax.axis_index("c")
    s = lax.axis_index("s")
    w = s * NC + c
    pltpu.sync_copy(src_hbm.at[pl.ds(w * CPW, QC)], sidx0)
    pltpu.sync_copy(dst_hbm.at[pl.ds(w * CPW, QC)], didx0)

    def fill_z(i, _):
        for j in range(H // 16):
            zbuf[i, pl.ds(j * 16, 16)] = jnp.zeros((16,), jnp.float32)
        return 0

    lax.fori_loop(0, 32, fill_z, 0)

    def zero_sh(i, _):
        pltpu.sync_copy(zbuf, agg_sh.at[pl.ds(s * RPS + i * 32, 32)])
        return 0

    lax.fori_loop(0, RPS // 32, zero_sh, 0)
    plsc.subcore_barrier()

    bufs = [(sidx0, didx0), (sidx1, didx1)]
    for q in range(NQ):
        sidx, didx = bufs[q % 2]
        nsidx, ndidx = bufs[(q + 1) % 2]
        if q < NQ - 1:
            off = w * CPW + (q + 1) * QC
            pltpu.async_copy(src_hbm.at[pl.ds(off, QC)], nsidx, sem_i)
            pltpu.async_copy(dst_hbm.at[pl.ds(off, QC)], ndidx, sem_i)

        def pair(g, _, sidx=sidx, didx=didx):
            i0 = 2 * g
            pltpu.async_copy(h_hbm.at[sidx.at[i0 + 1]], rows_b, sem_b)
            pltpu.make_async_copy(h_hbm.at[sidx.at[i0]], rows_a, sem_a).wait()
            pltpu.sync_copy(rows_a, agg_sh.at[didx.at[i0]], add=True)

            @pl.when(g < QPAIRS - 1)
            def _():
                pltpu.async_copy(h_hbm.at[sidx.at[i0 + 2]], rows_a, sem_a)

            pltpu.make_async_copy(h_hbm.at[sidx.at[i0 + 1]], rows_b, sem_b).wait()
            pltpu.sync_copy(rows_b, agg_sh.at[didx.at[i0 + 1]], add=True)
            return 0

        pltpu.async_copy(h_hbm.at[sidx.at[0]], rows_a, sem_a)
        lax.fori_loop(0, QPAIRS, pair, 0)
        if q < NQ - 1:
            pltpu.make_async_copy(src_hbm.at[pl.ds(0, QC)], nsidx, sem_i).wait()
            pltpu.make_async_copy(dst_hbm.at[pl.ds(0, QC)], ndidx, sem_i).wait()
    plsc.subcore_barrier()
    pltpu.sync_copy(
        agg_sh.at[pl.ds(s * RPS, RPS)], out_hbm.at[c, pl.ds(s * RPS, RPS)]
    )


def _input_body(x_ref, w_ref, b_ref, o_ref):
    o_ref[...] = jnp.maximum(
        jnp.dot(x_ref[...], w_ref[...], preferred_element_type=jnp.float32)
        + b_ref[...][None, :],
        0.0,
    )


def _layer_body(p_ref, deg_ref, h_ref, wl_ref, wr_ref, bl_ref, g_ref, bt_ref, o_ref):
    agg = p_ref[0, :N] + p_ref[1, :N]
    mean = agg / jnp.maximum(deg_ref[...], 1.0)
    h = h_ref[...]
    hn = (
        jnp.dot(mean, wl_ref[...], preferred_element_type=jnp.float32)
        + bl_ref[...][None, :]
        + jnp.dot(h, wr_ref[...], preferred_element_type=jnp.float32)
    )
    mu = jnp.mean(hn, axis=0, keepdims=True)
    var = jnp.mean((hn - mu) * (hn - mu), axis=0, keepdims=True)
    hb = (hn - mu) * lax.rsqrt(var + 1e-5) * g_ref[...][None, :] + bt_ref[...][None, :]
    o_ref[...] = jnp.maximum(hb, 0.0) + h


def _final_body(
    p_ref, deg_ref, h_ref, wl_ref, wr_ref, bl_ref,
    wn1_ref, bn1_ref, wn2_ref, bn2_ref,
    wg1_ref, bg1_ref, wg2_ref, bg2_ref,
    ne_ref, ge_ref,
):
    agg = p_ref[0, :N] + p_ref[1, :N]
    mean = agg / jnp.maximum(deg_ref[...], 1.0)
    h3 = (
        jnp.dot(mean, wl_ref[...], preferred_element_type=jnp.float32)
        + bl_ref[...][None, :]
        + jnp.dot(h_ref[...], wr_ref[...], preferred_element_type=jnp.float32)
    )
    t = jnp.maximum(
        jnp.dot(h3, wn1_ref[...], preferred_element_type=jnp.float32)
        + bn1_ref[...][None, :],
        0.0,
    )
    ne = jnp.dot(t, wn2_ref[...], preferred_element_type=jnp.float32) + bn2_ref[...][None, :]
    ne_ref[...] = ne
    gm = jnp.mean(ne, axis=0, keepdims=True)
    gx = jnp.max(ne, axis=0, keepdims=True)
    g = jnp.concatenate([gm, gx], axis=1)
    gh = jnp.maximum(
        jnp.dot(g, wg1_ref[...], preferred_element_type=jnp.float32)
        + bg1_ref[...][None, :],
        0.0,
    )
    ge_ref[...] = (
        jnp.dot(gh, wg2_ref[...], preferred_element_type=jnp.float32)
        + bg2_ref[...][None, :]
    )


def kernel(x, edge_index, edge_attr, W_in, b_in, Wl0, Wr0, bl0, Wl1, Wr1, bl1,
           Wl2, Wr2, bl2, Wl3, Wr3, bl3, gamma0, beta0, gamma1, beta1,
           gamma2, beta2, Wn1, bn1, Wn2, bn2, Wg1, bg1, Wg2, bg2):
    npad = EP - E
    pad_src = (jnp.arange(npad, dtype=jnp.int32) * 37) % N
    pad_dst = N + (jnp.arange(npad, dtype=jnp.int32) % (NP - N))
    src = jnp.concatenate([edge_index[0], pad_src]).reshape(NW * CPW, K)
    dst = jnp.concatenate([edge_index[1], pad_dst]).reshape(NW * CPW, K)

    h = pl.pallas_call(
        _input_body,
        out_shape=jax.ShapeDtypeStruct((N, H), jnp.float32),
    )(x, W_in, b_in)

    dd = _sc_degree(edge_index[1])
    deg = (dd[0] + dd[1])[:N, None]

    layer = pl.pallas_call(
        _layer_body,
        out_shape=jax.ShapeDtypeStruct((N, H), jnp.float32),
    )

    for Wl, Wr, bl, gm, bt in (
        (Wl0, Wr0, bl0, gamma0, beta0),
        (Wl1, Wr1, bl1, gamma1, beta1),
        (Wl2, Wr2, bl2, gamma2, beta2),
    ):
        p = _sc_agg(h, src, dst)
        h = layer(p, deg, h, Wl, Wr, bl, gm, bt)

    p = _sc_agg(h, src, dst)
    node_emb, graph_emb = pl.pallas_call(
        _final_body,
        out_shape=(
            jax.ShapeDtypeStruct((N, OUT), jnp.float32),
            jax.ShapeDtypeStruct((1, OUT), jnp.float32),
        ),
    )(p, deg, h, Wl3, Wr3, bl3, Wn1, bn1, Wn2, bn2, Wg1, bg1, Wg2, bg2)

    return (node_emb, graph_emb.reshape(OUT))
